# Initial kernel scaffold; baseline (speedup 1.0000x reference)
import jax
import jax.numpy as jnp
from jax.experimental import pallas as pl


def kernel(x, token_table, pos_table):
    raise NotImplementedError("write your pallas kernel here")


# speedup vs baseline: 3.7202x; 3.7202x over previous
import functools

import jax
import jax.numpy as jnp
from jax import lax
from jax.experimental import pallas as pl
from jax.experimental.pallas import tpu as pltpu
from jax.experimental.pallas import tpu_sc as plsc

D = 64
S = 200
NC, NS = 2, 16
NW = NC * NS

CHUNK_SEQ = 4
CHUNK_ROWS = CHUNK_SEQ * S
GATHER_SLICE = 128


def _emb_sc(x_flat, token_table, pos_table):
  n_rows = x_flat.shape[0]
  rows_per_w = n_rows // NW
  steps = rows_per_w // CHUNK_ROWS

  mesh = plsc.VectorSubcoreMesh(core_axis_name="c", subcore_axis_name="s")

  @functools.partial(
      pl.kernel,
      out_type=jax.ShapeDtypeStruct((n_rows, D), jnp.float32),
      mesh=mesh,
      scratch_types=[
          pltpu.VMEM((CHUNK_ROWS,), jnp.int32),
          pltpu.VMEM((CHUNK_ROWS, D), jnp.float32),
          pltpu.VMEM_SHARED((S, D), jnp.float32),
          pltpu.SemaphoreType.DMA,
      ],
      compiler_params=pltpu.CompilerParams(use_tc_tiling_on_sc=False),
  )
  def k(x_hbm, tok_hbm, pos_hbm, out_hbm, idx_v, buf_v, pos_sh, sem):
    sid = lax.axis_index("s")
    wid = sid * NC + lax.axis_index("c")

    @pl.when(sid == 0)
    def _():
      pltpu.sync_copy(pos_hbm, buf_v.at[pl.ds(0, S), :])
      pltpu.sync_copy(buf_v.at[pl.ds(0, S), :], pos_sh)

    plsc.subcore_barrier()
    w_base = wid * rows_per_w

    def step(c, carry):
      base = w_base + c * CHUNK_ROWS
      pltpu.sync_copy(x_hbm.at[pl.ds(base, CHUNK_ROWS)], idx_v)
      for s_i in range(CHUNK_SEQ):
        pltpu.sync_copy(pos_sh, buf_v.at[pl.ds(s_i * S, S), :])
      descs = []
      off = 0
      while off < CHUNK_ROWS:
        n = min(GATHER_SLICE, CHUNK_ROWS - off)
        descs.append(
            pltpu.async_copy(
                tok_hbm.at[idx_v.at[pl.ds(off, n)]],
                buf_v.at[pl.ds(off, n), :],
                sem,
                add=True,
            )
        )
        off += n
      for d_ in descs:
        d_.wait()
      pltpu.sync_copy(buf_v, out_hbm.at[pl.ds(base, CHUNK_ROWS), :])
      return carry

    lax.fori_loop(0, steps, step, 0)

  return k(x_flat, token_table, pos_table)


def kernel(x, token_table, pos_table):
  b, s = x.shape
  x_flat = x.reshape(-1).astype(jnp.int32)
  out = _emb_sc(x_flat, token_table.astype(jnp.float32),
                pos_table.astype(jnp.float32))
  return out.reshape(b, s, token_table.shape[1])


# speedup vs baseline: 4.1660x; 1.1198x over previous
import functools

import jax
import jax.numpy as jnp
from jax import lax
from jax.experimental import pallas as pl
from jax.experimental.pallas import tpu as pltpu
from jax.experimental.pallas import tpu_sc as plsc

D = 64
S = 200
NC, NS = 2, 16
NW = NC * NS

CHUNK_SEQ = 4
CHUNK_ROWS = CHUNK_SEQ * S
GATHER_SLICE = 128


def _emb_sc(x_flat, token_table, pos_table):
  n_rows = x_flat.shape[0]
  rows_per_w = n_rows // NW
  steps = rows_per_w // CHUNK_ROWS

  mesh = plsc.VectorSubcoreMesh(core_axis_name="c", subcore_axis_name="s")

  @functools.partial(
      pl.kernel,
      out_type=jax.ShapeDtypeStruct((n_rows, D), jnp.float32),
      mesh=mesh,
      scratch_types=[
          pltpu.VMEM((CHUNK_ROWS,), jnp.int32),
          pltpu.VMEM((CHUNK_ROWS,), jnp.int32),
          pltpu.VMEM((CHUNK_ROWS, D), jnp.float32),
          pltpu.VMEM((CHUNK_ROWS, D), jnp.float32),
          pltpu.VMEM_SHARED((S, D), jnp.float32),
          pltpu.SemaphoreType.DMA,
          pltpu.SemaphoreType.DMA,
          pltpu.SemaphoreType.DMA,
      ],
      compiler_params=pltpu.CompilerParams(use_tc_tiling_on_sc=False),
  )
  def k(x_hbm, tok_hbm, pos_hbm, out_hbm, idx0, idx1, buf0, buf1, pos_sh,
        sem_pre, sem_g, sem_wb):
    idxs = (idx0, idx1)
    bufs = (buf0, buf1)
    sid = lax.axis_index("s")
    wid = sid * NC + lax.axis_index("c")

    @pl.when(sid == 0)
    def _():
      pltpu.sync_copy(pos_hbm, buf0.at[pl.ds(0, S), :])
      pltpu.sync_copy(buf0.at[pl.ds(0, S), :], pos_sh)

    plsc.subcore_barrier()
    w_base = wid * rows_per_w

    def stage_and_prefill(c, b):
      base = w_base + c * CHUNK_ROWS
      pltpu.sync_copy(x_hbm.at[pl.ds(base, CHUNK_ROWS)], idxs[b])
      for s_i in range(CHUNK_SEQ):
        pltpu.async_copy(pos_sh, bufs[b].at[pl.ds(s_i * S, S), :], sem_pre)

    def chunk_body(c, b):
      b1 = 1 - b
      for s_i in range(CHUNK_SEQ):
        pltpu.make_async_copy(
            pos_sh, bufs[b].at[pl.ds(s_i * S, S), :], sem_pre).wait()
      descs = []
      off = 0
      while off < CHUNK_ROWS:
        n = min(GATHER_SLICE, CHUNK_ROWS - off)
        descs.append(
            pltpu.async_copy(
                tok_hbm.at[idxs[b].at[pl.ds(off, n)]],
                bufs[b].at[pl.ds(off, n), :],
                sem_g,
                add=True,
            )
        )
        off += n
      @pl.when(c + 1 < steps)
      def _():
        @pl.when(c >= 1)
        def _():
          pltpu.make_async_copy(
              bufs[b1], out_hbm.at[pl.ds(w_base, CHUNK_ROWS), :],
              sem_wb).wait()
        stage_and_prefill(c + 1, b1)
      for d_ in descs:
        d_.wait()
      base = w_base + c * CHUNK_ROWS
      pltpu.async_copy(bufs[b], out_hbm.at[pl.ds(base, CHUNK_ROWS), :],
                       sem_wb)

    stage_and_prefill(0, 0)

    def outer(c2, carry):
      for b in range(2):
        chunk_body(c2 * 2 + b, b)
      return carry

    lax.fori_loop(0, steps // 2, outer, 0)

    for b in range(2):
      pltpu.make_async_copy(
          bufs[b], out_hbm.at[pl.ds(w_base, CHUNK_ROWS), :], sem_wb).wait()

  return k(x_flat, token_table, pos_table)


def kernel(x, token_table, pos_table):
  b, s = x.shape
  x_flat = x.reshape(-1).astype(jnp.int32)
  out = _emb_sc(x_flat, token_table.astype(jnp.float32),
                pos_table.astype(jnp.float32))
  return out.reshape(b, s, token_table.shape[1])
